# baseline (speedup 1.0000x reference)
import functools

import jax
import jax.numpy as jnp
import numpy as np
from jax.experimental import pallas as pl
from jax.sharding import Mesh, PartitionSpec as P


def _safe(x):
    eps = 1e-12
    return jnp.where(jnp.abs(x) < eps, jnp.where(x < 0, -eps, eps), x)


def _sweep_kernel(tri_ref, tri2_ref, pts_ref, res_ref, ct_ref, *, F, FT, QT):
    pts = pts_ref[0]
    px, py, pz = pts[:, 0:1], pts[:, 1:2], pts[:, 2:3]

    nft = F // FT
    neg = jnp.float32(0.0)
    inf = jnp.float32(3.4e38)

    def body(i, carry):
        best_d2, best_vals = carry
        t = tri_ref[0, :, pl.ds(i * FT, FT)]
        ax, ay, az = t[0:1], t[1:2], t[2:3]
        bx, by, bz = t[3:4], t[4:5], t[5:6]
        cx, cy, cz = t[6:7], t[7:8], t[8:9]

        abx, aby, abz = bx - ax, by - ay, bz - az
        acx, acy, acz = cx - ax, cy - ay, cz - az

        apx, apy, apz = px - ax, py - ay, pz - az
        bpx, bpy, bpz = px - bx, py - by, pz - bz
        cpx, cpy, cpz = px - cx, py - cy, pz - cz
        d1 = abx * apx + aby * apy + abz * apz
        d2 = acx * apx + acy * apy + acz * apz
        d3 = abx * bpx + aby * bpy + abz * bpz
        d4 = acx * bpx + acy * bpy + acz * bpz
        d5 = abx * cpx + aby * cpy + abz * cpz
        d6 = acx * cpx + acy * cpy + acz * cpz

        vc = d1 * d4 - d3 * d2
        vb = d5 * d2 - d1 * d6
        va = d3 * d6 - d5 * d4
        v_ab = d1 / _safe(d1 - d3)
        w_ac = d2 / _safe(d2 - d6)
        w_bc = (d4 - d3) / _safe((d4 - d3) + (d5 - d6))
        denom = _safe(va + vb + vc)
        v_in = vb / denom
        w_in = vc / denom

        u = 1.0 - v_in - w_in
        v = v_in
        w = w_in
        cond = (va <= 0) & ((d4 - d3) >= 0) & ((d5 - d6) >= 0)
        u = jnp.where(cond, 0.0, u)
        v = jnp.where(cond, 1.0 - w_bc, v)
        w = jnp.where(cond, w_bc, w)
        cond = (vb <= 0) & (d2 >= 0) & (d6 <= 0)
        u = jnp.where(cond, 1.0 - w_ac, u)
        v = jnp.where(cond, 0.0, v)
        w = jnp.where(cond, w_ac, w)
        cond = (d6 >= 0) & (d5 <= d6)
        u = jnp.where(cond, 0.0, u)
        v = jnp.where(cond, 0.0, v)
        w = jnp.where(cond, 1.0, w)
        cond = (vc <= 0) & (d1 >= 0) & (d3 <= 0)
        u = jnp.where(cond, 1.0 - v_ab, u)
        v = jnp.where(cond, v_ab, v)
        w = jnp.where(cond, 0.0, w)
        cond = (d3 >= 0) & (d4 <= d3)
        u = jnp.where(cond, 0.0, u)
        v = jnp.where(cond, 1.0, v)
        w = jnp.where(cond, 0.0, w)
        cond = (d1 <= 0) & (d2 <= 0)
        u = jnp.where(cond, 1.0, u)
        v = jnp.where(cond, 0.0, v)
        w = jnp.where(cond, 0.0, w)

        clx = u * ax + v * bx + w * cx
        cly = u * ay + v * by + w * cy
        clz = u * az + v * bz + w * cz
        dx = px - clx
        dy = py - cly
        dz = pz - clz
        dist2 = dx * dx + dy * dy + dz * dz

        tmin = jnp.min(dist2, axis=1, keepdims=True)
        lane = jax.lax.broadcasted_iota(jnp.int32, (QT, FT), 1)
        cand = jnp.where(dist2 == tmin, lane, jnp.int32(2**30))
        tidx = jnp.min(cand, axis=1, keepdims=True)
        onehot = lane == tidx

        onef = jnp.where(onehot, 1.0, neg).astype(jnp.bfloat16)
        t2 = tri2_ref[0, pl.ds(i * FT, FT), :]
        t2hi = t2.astype(jnp.bfloat16)
        t2lo = (t2 - t2hi.astype(jnp.float32)).astype(jnp.bfloat16)
        dn = (((1,), (0,)), ((), ()))
        vals = (jax.lax.dot_general(onef, t2hi, dn,
                                    preferred_element_type=jnp.float32)
                + jax.lax.dot_general(onef, t2lo, dn,
                                      preferred_element_type=jnp.float32))

        improved = tmin < best_d2
        best_d2 = jnp.where(improved, tmin, best_d2)
        best_vals = jnp.where(improved, vals, best_vals)
        return best_d2, best_vals

    init = (jnp.full((QT, 1), inf, jnp.float32),
            jnp.zeros((QT, 9), jnp.float32))
    _, best_vals = jax.lax.fori_loop(0, nft, body, init)

    tri9 = best_vals
    wax, way, waz = tri9[:, 0:1], tri9[:, 1:2], tri9[:, 2:3]
    wbx, wby, wbz = tri9[:, 3:4], tri9[:, 4:5], tri9[:, 5:6]
    wcx, wcy, wcz = tri9[:, 6:7], tri9[:, 7:8], tri9[:, 8:9]
    abx, aby, abz = wbx - wax, wby - way, wbz - waz
    acx, acy, acz = wcx - wax, wcy - way, wcz - waz
    apx, apy, apz = px - wax, py - way, pz - waz
    bpx, bpy, bpz = px - wbx, py - wby, pz - wbz
    cpx, cpy, cpz = px - wcx, py - wcy, pz - wcz
    d1 = abx * apx + aby * apy + abz * apz
    d2 = acx * apx + acy * apy + acz * apz
    d3 = abx * bpx + aby * bpy + abz * bpz
    d4 = acx * bpx + acy * bpy + acz * bpz
    d5 = abx * cpx + aby * cpy + abz * cpz
    d6 = acx * cpx + acy * cpy + acz * cpz
    vc = d1 * d4 - d3 * d2
    vb = d5 * d2 - d1 * d6
    va = d3 * d6 - d5 * d4
    v_ab = d1 / _safe(d1 - d3)
    w_ac = d2 / _safe(d2 - d6)
    w_bc = (d4 - d3) / _safe((d4 - d3) + (d5 - d6))
    denom = _safe(va + vb + vc)
    v_in = vb / denom
    w_in = vc / denom
    u = 1.0 - v_in - w_in
    v = v_in
    w = w_in
    cond = (va <= 0) & ((d4 - d3) >= 0) & ((d5 - d6) >= 0)
    u = jnp.where(cond, 0.0, u)
    v = jnp.where(cond, 1.0 - w_bc, v)
    w = jnp.where(cond, w_bc, w)
    cond = (vb <= 0) & (d2 >= 0) & (d6 <= 0)
    u = jnp.where(cond, 1.0 - w_ac, u)
    v = jnp.where(cond, 0.0, v)
    w = jnp.where(cond, w_ac, w)
    cond = (d6 >= 0) & (d5 <= d6)
    u = jnp.where(cond, 0.0, u)
    v = jnp.where(cond, 0.0, v)
    w = jnp.where(cond, 1.0, w)
    cond = (vc <= 0) & (d1 >= 0) & (d3 <= 0)
    u = jnp.where(cond, 1.0 - v_ab, u)
    v = jnp.where(cond, v_ab, v)
    w = jnp.where(cond, 0.0, w)
    cond = (d3 >= 0) & (d4 <= d3)
    u = jnp.where(cond, 0.0, u)
    v = jnp.where(cond, 1.0, v)
    w = jnp.where(cond, 0.0, w)
    cond = (d1 <= 0) & (d2 <= 0)
    u = jnp.where(cond, 1.0, u)
    v = jnp.where(cond, 0.0, v)
    w = jnp.where(cond, 0.0, w)

    u = jnp.clip(u, 0.0, 1.0)
    v = jnp.clip(v, 0.0, 1.0)
    w = jnp.clip(w, 0.0, 1.0)
    clx = u * tri9[:, 0:1] + v * tri9[:, 3:4] + w * tri9[:, 6:7]
    cly = u * tri9[:, 1:2] + v * tri9[:, 4:5] + w * tri9[:, 7:8]
    clz = u * tri9[:, 2:3] + v * tri9[:, 5:6] + w * tri9[:, 8:9]
    res_ref[0] = jnp.concatenate([clx - px, cly - py, clz - pz], axis=1)
    ct_ref[0] = tri9


def _run(triangles, points, interpret=False):
    B, F = triangles.shape[0], triangles.shape[1]
    Q = points.shape[1]
    QT = 128
    FT = 512
    tri2 = triangles.reshape(B, F, 9)
    tri = tri2.transpose(0, 2, 1)
    res, ct = pl.pallas_call(
        functools.partial(_sweep_kernel, F=F, FT=FT, QT=QT),
        grid=(B, Q // QT),
        in_specs=[
            pl.BlockSpec((1, 9, F), lambda b, q: (b, 0, 0)),
            pl.BlockSpec((1, F, 9), lambda b, q: (b, 0, 0)),
            pl.BlockSpec((1, QT, 3), lambda b, q: (b, q, 0)),
        ],
        out_specs=[
            pl.BlockSpec((1, QT, 3), lambda b, q: (b, q, 0)),
            pl.BlockSpec((1, QT, 9), lambda b, q: (b, q, 0)),
        ],
        out_shape=[
            jax.ShapeDtypeStruct((B, Q, 3), jnp.float32),
            jax.ShapeDtypeStruct((B, Q, 9), jnp.float32),
        ],
        interpret=interpret,
    )(tri, tri2, points)
    return res, ct.reshape(B, Q, 3, 3)


_shard_map = getattr(jax, "shard_map", None)
if _shard_map is None:
    from jax.experimental.shard_map import shard_map as _shard_map


@jax.jit
def _run_sharded(triangles, points):
    B = triangles.shape[0]
    devs = jax.devices()
    n = max(d for d in range(1, min(len(devs), B) + 1) if B % d == 0)
    if n == 1:
        return _run(triangles, points)
    mesh = Mesh(np.array(devs[:n]), ("d",))
    return _shard_map(
        _run,
        mesh=mesh,
        in_specs=(P("d"), P("d")),
        out_specs=(P("d"), P("d")),
        check_vma=False,
    )(triangles, points)


def kernel(triangles, points):
    return _run_sharded(triangles, points)


# speedup vs baseline: 1.1288x; 1.1288x over previous
import functools

import jax
import jax.numpy as jnp
import numpy as np
from jax.experimental import pallas as pl
from jax.sharding import Mesh, PartitionSpec as P


def _safe(x):
    eps = 1e-12
    return jnp.where(jnp.abs(x) < eps, jnp.where(x < 0, -eps, eps), x)


def _sweep_kernel(tri_ref, tri2_ref, pts_ref, res_ref, ct_ref, *, F, FT, QT):
    pts = pts_ref[0]
    px, py, pz = pts[:, 0:1], pts[:, 1:2], pts[:, 2:3]

    nft = F // FT
    neg = jnp.float32(0.0)
    inf = jnp.float32(3.4e38)

    def body(i, carry):
        best_d2, best_vals = carry
        t = tri_ref[0, :, pl.ds(i * FT, FT)]
        ax, ay, az = t[0:1], t[1:2], t[2:3]
        bx, by, bz = t[3:4], t[4:5], t[5:6]
        cx, cy, cz = t[6:7], t[7:8], t[8:9]

        abx, aby, abz = bx - ax, by - ay, bz - az
        acx, acy, acz = cx - ax, cy - ay, cz - az

        apx, apy, apz = px - ax, py - ay, pz - az
        bpx, bpy, bpz = px - bx, py - by, pz - bz
        cpx, cpy, cpz = px - cx, py - cy, pz - cz
        d1 = abx * apx + aby * apy + abz * apz
        d2 = acx * apx + acy * apy + acz * apz
        d3 = abx * bpx + aby * bpy + abz * bpz
        d4 = acx * bpx + acy * bpy + acz * bpz
        d5 = abx * cpx + aby * cpy + abz * cpz
        d6 = acx * cpx + acy * cpy + acz * cpz

        vc = d1 * d4 - d3 * d2
        vb = d5 * d2 - d1 * d6
        va = d3 * d6 - d5 * d4
        v_ab = d1 / _safe(d1 - d3)
        w_ac = d2 / _safe(d2 - d6)
        w_bc = (d4 - d3) / _safe((d4 - d3) + (d5 - d6))
        denom = _safe(va + vb + vc)
        v_in = vb / denom
        w_in = vc / denom

        u = 1.0 - v_in - w_in
        v = v_in
        w = w_in
        cond = (va <= 0) & ((d4 - d3) >= 0) & ((d5 - d6) >= 0)
        u = jnp.where(cond, 0.0, u)
        v = jnp.where(cond, 1.0 - w_bc, v)
        w = jnp.where(cond, w_bc, w)
        cond = (vb <= 0) & (d2 >= 0) & (d6 <= 0)
        u = jnp.where(cond, 1.0 - w_ac, u)
        v = jnp.where(cond, 0.0, v)
        w = jnp.where(cond, w_ac, w)
        cond = (d6 >= 0) & (d5 <= d6)
        u = jnp.where(cond, 0.0, u)
        v = jnp.where(cond, 0.0, v)
        w = jnp.where(cond, 1.0, w)
        cond = (vc <= 0) & (d1 >= 0) & (d3 <= 0)
        u = jnp.where(cond, 1.0 - v_ab, u)
        v = jnp.where(cond, v_ab, v)
        w = jnp.where(cond, 0.0, w)
        cond = (d3 >= 0) & (d4 <= d3)
        u = jnp.where(cond, 0.0, u)
        v = jnp.where(cond, 1.0, v)
        w = jnp.where(cond, 0.0, w)
        cond = (d1 <= 0) & (d2 <= 0)
        u = jnp.where(cond, 1.0, u)
        v = jnp.where(cond, 0.0, v)
        w = jnp.where(cond, 0.0, w)

        clx = u * ax + v * bx + w * cx
        cly = u * ay + v * by + w * cy
        clz = u * az + v * bz + w * cz
        dx = px - clx
        dy = py - cly
        dz = pz - clz
        dist2 = dx * dx + dy * dy + dz * dz

        tmin = jnp.min(dist2, axis=1, keepdims=True)
        lane = jax.lax.broadcasted_iota(jnp.int32, (QT, FT), 1)
        cand = jnp.where(dist2 == tmin, lane, jnp.int32(2**30))
        tidx = jnp.min(cand, axis=1, keepdims=True)
        onehot = lane == tidx

        onef = jnp.where(onehot, 1.0, neg).astype(jnp.bfloat16)
        t2 = tri2_ref[0, pl.ds(i * FT, FT), :]
        t2hi = t2.astype(jnp.bfloat16)
        t2lo = (t2 - t2hi.astype(jnp.float32)).astype(jnp.bfloat16)
        dn = (((1,), (0,)), ((), ()))
        vals = (jax.lax.dot_general(onef, t2hi, dn,
                                    preferred_element_type=jnp.float32)
                + jax.lax.dot_general(onef, t2lo, dn,
                                      preferred_element_type=jnp.float32))

        improved = tmin < best_d2
        best_d2 = jnp.where(improved, tmin, best_d2)
        best_vals = jnp.where(improved, vals, best_vals)
        return best_d2, best_vals

    init = (jnp.full((QT, 1), inf, jnp.float32),
            jnp.zeros((QT, 9), jnp.float32))
    _, best_vals = jax.lax.fori_loop(0, nft, body, init)

    tri9 = best_vals
    wax, way, waz = tri9[:, 0:1], tri9[:, 1:2], tri9[:, 2:3]
    wbx, wby, wbz = tri9[:, 3:4], tri9[:, 4:5], tri9[:, 5:6]
    wcx, wcy, wcz = tri9[:, 6:7], tri9[:, 7:8], tri9[:, 8:9]
    abx, aby, abz = wbx - wax, wby - way, wbz - waz
    acx, acy, acz = wcx - wax, wcy - way, wcz - waz
    apx, apy, apz = px - wax, py - way, pz - waz
    bpx, bpy, bpz = px - wbx, py - wby, pz - wbz
    cpx, cpy, cpz = px - wcx, py - wcy, pz - wcz
    d1 = abx * apx + aby * apy + abz * apz
    d2 = acx * apx + acy * apy + acz * apz
    d3 = abx * bpx + aby * bpy + abz * bpz
    d4 = acx * bpx + acy * bpy + acz * bpz
    d5 = abx * cpx + aby * cpy + abz * cpz
    d6 = acx * cpx + acy * cpy + acz * cpz
    vc = d1 * d4 - d3 * d2
    vb = d5 * d2 - d1 * d6
    va = d3 * d6 - d5 * d4
    v_ab = d1 / _safe(d1 - d3)
    w_ac = d2 / _safe(d2 - d6)
    w_bc = (d4 - d3) / _safe((d4 - d3) + (d5 - d6))
    denom = _safe(va + vb + vc)
    v_in = vb / denom
    w_in = vc / denom
    u = 1.0 - v_in - w_in
    v = v_in
    w = w_in
    cond = (va <= 0) & ((d4 - d3) >= 0) & ((d5 - d6) >= 0)
    u = jnp.where(cond, 0.0, u)
    v = jnp.where(cond, 1.0 - w_bc, v)
    w = jnp.where(cond, w_bc, w)
    cond = (vb <= 0) & (d2 >= 0) & (d6 <= 0)
    u = jnp.where(cond, 1.0 - w_ac, u)
    v = jnp.where(cond, 0.0, v)
    w = jnp.where(cond, w_ac, w)
    cond = (d6 >= 0) & (d5 <= d6)
    u = jnp.where(cond, 0.0, u)
    v = jnp.where(cond, 0.0, v)
    w = jnp.where(cond, 1.0, w)
    cond = (vc <= 0) & (d1 >= 0) & (d3 <= 0)
    u = jnp.where(cond, 1.0 - v_ab, u)
    v = jnp.where(cond, v_ab, v)
    w = jnp.where(cond, 0.0, w)
    cond = (d3 >= 0) & (d4 <= d3)
    u = jnp.where(cond, 0.0, u)
    v = jnp.where(cond, 1.0, v)
    w = jnp.where(cond, 0.0, w)
    cond = (d1 <= 0) & (d2 <= 0)
    u = jnp.where(cond, 1.0, u)
    v = jnp.where(cond, 0.0, v)
    w = jnp.where(cond, 0.0, w)

    u = jnp.clip(u, 0.0, 1.0)
    v = jnp.clip(v, 0.0, 1.0)
    w = jnp.clip(w, 0.0, 1.0)
    clx = u * tri9[:, 0:1] + v * tri9[:, 3:4] + w * tri9[:, 6:7]
    cly = u * tri9[:, 1:2] + v * tri9[:, 4:5] + w * tri9[:, 7:8]
    clz = u * tri9[:, 2:3] + v * tri9[:, 5:6] + w * tri9[:, 8:9]
    res_ref[0] = jnp.concatenate([clx - px, cly - py, clz - pz], axis=1)
    ct_ref[0] = tri9


def _run(triangles, points, interpret=False):
    B, F = triangles.shape[0], triangles.shape[1]
    Q = points.shape[1]
    QT = 512
    FT = 512
    tri2 = triangles.reshape(B, F, 9)
    tri = tri2.transpose(0, 2, 1)
    res, ct = pl.pallas_call(
        functools.partial(_sweep_kernel, F=F, FT=FT, QT=QT),
        grid=(B, Q // QT),
        in_specs=[
            pl.BlockSpec((1, 9, F), lambda b, q: (b, 0, 0)),
            pl.BlockSpec((1, F, 9), lambda b, q: (b, 0, 0)),
            pl.BlockSpec((1, QT, 3), lambda b, q: (b, q, 0)),
        ],
        out_specs=[
            pl.BlockSpec((1, QT, 3), lambda b, q: (b, q, 0)),
            pl.BlockSpec((1, QT, 9), lambda b, q: (b, q, 0)),
        ],
        out_shape=[
            jax.ShapeDtypeStruct((B, Q, 3), jnp.float32),
            jax.ShapeDtypeStruct((B, Q, 9), jnp.float32),
        ],
        interpret=interpret,
    )(tri, tri2, points)
    return res, ct.reshape(B, Q, 3, 3)


_shard_map = getattr(jax, "shard_map", None)
if _shard_map is None:
    from jax.experimental.shard_map import shard_map as _shard_map


@jax.jit
def _run_sharded(triangles, points):
    B = triangles.shape[0]
    devs = jax.devices()
    n = max(d for d in range(1, min(len(devs), B) + 1) if B % d == 0)
    if n == 1:
        return _run(triangles, points)
    mesh = Mesh(np.array(devs[:n]), ("d",))
    return _shard_map(
        _run,
        mesh=mesh,
        in_specs=(P("d"), P("d")),
        out_specs=(P("d"), P("d")),
        check_vma=False,
    )(triangles, points)


def kernel(triangles, points):
    return _run_sharded(triangles, points)


# speedup vs baseline: 1.1357x; 1.0061x over previous
import functools

import jax
import jax.numpy as jnp
import numpy as np
from jax.experimental import pallas as pl
from jax.experimental.pallas import tpu as pltpu
from jax.sharding import Mesh, PartitionSpec as P


def _safe(x):
    eps = 1e-12
    return jnp.where(jnp.abs(x) < eps, jnp.where(x < 0, -eps, eps), x)


def _sweep_kernel(tri_ref, tri2_ref, pts_ref, res_ref, ct_ref, *, F, FT, QT):
    pts = pts_ref[0]
    px, py, pz = pts[:, 0:1], pts[:, 1:2], pts[:, 2:3]

    nft = F // FT
    neg = jnp.float32(0.0)
    inf = jnp.float32(3.4e38)

    def body(i, carry):
        best_d2, best_vals = carry
        t = tri_ref[0, :, pl.ds(i * FT, FT)]
        ax, ay, az = t[0:1], t[1:2], t[2:3]
        bx, by, bz = t[3:4], t[4:5], t[5:6]
        cx, cy, cz = t[6:7], t[7:8], t[8:9]

        abx, aby, abz = bx - ax, by - ay, bz - az
        acx, acy, acz = cx - ax, cy - ay, cz - az

        apx, apy, apz = px - ax, py - ay, pz - az
        bpx, bpy, bpz = px - bx, py - by, pz - bz
        cpx, cpy, cpz = px - cx, py - cy, pz - cz
        d1 = abx * apx + aby * apy + abz * apz
        d2 = acx * apx + acy * apy + acz * apz
        d3 = abx * bpx + aby * bpy + abz * bpz
        d4 = acx * bpx + acy * bpy + acz * bpz
        d5 = abx * cpx + aby * cpy + abz * cpz
        d6 = acx * cpx + acy * cpy + acz * cpz

        vc = d1 * d4 - d3 * d2
        vb = d5 * d2 - d1 * d6
        va = d3 * d6 - d5 * d4
        v_ab = d1 / _safe(d1 - d3)
        w_ac = d2 / _safe(d2 - d6)
        w_bc = (d4 - d3) / _safe((d4 - d3) + (d5 - d6))
        denom = _safe(va + vb + vc)
        v_in = vb / denom
        w_in = vc / denom

        u = 1.0 - v_in - w_in
        v = v_in
        w = w_in
        cond = (va <= 0) & ((d4 - d3) >= 0) & ((d5 - d6) >= 0)
        u = jnp.where(cond, 0.0, u)
        v = jnp.where(cond, 1.0 - w_bc, v)
        w = jnp.where(cond, w_bc, w)
        cond = (vb <= 0) & (d2 >= 0) & (d6 <= 0)
        u = jnp.where(cond, 1.0 - w_ac, u)
        v = jnp.where(cond, 0.0, v)
        w = jnp.where(cond, w_ac, w)
        cond = (d6 >= 0) & (d5 <= d6)
        u = jnp.where(cond, 0.0, u)
        v = jnp.where(cond, 0.0, v)
        w = jnp.where(cond, 1.0, w)
        cond = (vc <= 0) & (d1 >= 0) & (d3 <= 0)
        u = jnp.where(cond, 1.0 - v_ab, u)
        v = jnp.where(cond, v_ab, v)
        w = jnp.where(cond, 0.0, w)
        cond = (d3 >= 0) & (d4 <= d3)
        u = jnp.where(cond, 0.0, u)
        v = jnp.where(cond, 1.0, v)
        w = jnp.where(cond, 0.0, w)
        cond = (d1 <= 0) & (d2 <= 0)
        u = jnp.where(cond, 1.0, u)
        v = jnp.where(cond, 0.0, v)
        w = jnp.where(cond, 0.0, w)

        clx = u * ax + v * bx + w * cx
        cly = u * ay + v * by + w * cy
        clz = u * az + v * bz + w * cz
        dx = px - clx
        dy = py - cly
        dz = pz - clz
        dist2 = dx * dx + dy * dy + dz * dz

        tmin = jnp.min(dist2, axis=1, keepdims=True)
        lane = jax.lax.broadcasted_iota(jnp.int32, (QT, FT), 1)
        cand = jnp.where(dist2 == tmin, lane, jnp.int32(2**30))
        tidx = jnp.min(cand, axis=1, keepdims=True)
        onehot = lane == tidx

        onef = jnp.where(onehot, 1.0, neg).astype(jnp.bfloat16)
        t2 = tri2_ref[0, pl.ds(i * FT, FT), :]
        t2hi = t2.astype(jnp.bfloat16)
        t2lo = (t2 - t2hi.astype(jnp.float32)).astype(jnp.bfloat16)
        dn = (((1,), (0,)), ((), ()))
        vals = (jax.lax.dot_general(onef, t2hi, dn,
                                    preferred_element_type=jnp.float32)
                + jax.lax.dot_general(onef, t2lo, dn,
                                      preferred_element_type=jnp.float32))

        improved = tmin < best_d2
        best_d2 = jnp.where(improved, tmin, best_d2)
        best_vals = jnp.where(improved, vals, best_vals)
        return best_d2, best_vals

    init = (jnp.full((QT, 1), inf, jnp.float32),
            jnp.zeros((QT, 9), jnp.float32))
    _, best_vals = jax.lax.fori_loop(0, nft, body, init)

    tri9 = best_vals
    wax, way, waz = tri9[:, 0:1], tri9[:, 1:2], tri9[:, 2:3]
    wbx, wby, wbz = tri9[:, 3:4], tri9[:, 4:5], tri9[:, 5:6]
    wcx, wcy, wcz = tri9[:, 6:7], tri9[:, 7:8], tri9[:, 8:9]
    abx, aby, abz = wbx - wax, wby - way, wbz - waz
    acx, acy, acz = wcx - wax, wcy - way, wcz - waz
    apx, apy, apz = px - wax, py - way, pz - waz
    bpx, bpy, bpz = px - wbx, py - wby, pz - wbz
    cpx, cpy, cpz = px - wcx, py - wcy, pz - wcz
    d1 = abx * apx + aby * apy + abz * apz
    d2 = acx * apx + acy * apy + acz * apz
    d3 = abx * bpx + aby * bpy + abz * bpz
    d4 = acx * bpx + acy * bpy + acz * bpz
    d5 = abx * cpx + aby * cpy + abz * cpz
    d6 = acx * cpx + acy * cpy + acz * cpz
    vc = d1 * d4 - d3 * d2
    vb = d5 * d2 - d1 * d6
    va = d3 * d6 - d5 * d4
    v_ab = d1 / _safe(d1 - d3)
    w_ac = d2 / _safe(d2 - d6)
    w_bc = (d4 - d3) / _safe((d4 - d3) + (d5 - d6))
    denom = _safe(va + vb + vc)
    v_in = vb / denom
    w_in = vc / denom
    u = 1.0 - v_in - w_in
    v = v_in
    w = w_in
    cond = (va <= 0) & ((d4 - d3) >= 0) & ((d5 - d6) >= 0)
    u = jnp.where(cond, 0.0, u)
    v = jnp.where(cond, 1.0 - w_bc, v)
    w = jnp.where(cond, w_bc, w)
    cond = (vb <= 0) & (d2 >= 0) & (d6 <= 0)
    u = jnp.where(cond, 1.0 - w_ac, u)
    v = jnp.where(cond, 0.0, v)
    w = jnp.where(cond, w_ac, w)
    cond = (d6 >= 0) & (d5 <= d6)
    u = jnp.where(cond, 0.0, u)
    v = jnp.where(cond, 0.0, v)
    w = jnp.where(cond, 1.0, w)
    cond = (vc <= 0) & (d1 >= 0) & (d3 <= 0)
    u = jnp.where(cond, 1.0 - v_ab, u)
    v = jnp.where(cond, v_ab, v)
    w = jnp.where(cond, 0.0, w)
    cond = (d3 >= 0) & (d4 <= d3)
    u = jnp.where(cond, 0.0, u)
    v = jnp.where(cond, 1.0, v)
    w = jnp.where(cond, 0.0, w)
    cond = (d1 <= 0) & (d2 <= 0)
    u = jnp.where(cond, 1.0, u)
    v = jnp.where(cond, 0.0, v)
    w = jnp.where(cond, 0.0, w)

    u = jnp.clip(u, 0.0, 1.0)
    v = jnp.clip(v, 0.0, 1.0)
    w = jnp.clip(w, 0.0, 1.0)
    clx = u * tri9[:, 0:1] + v * tri9[:, 3:4] + w * tri9[:, 6:7]
    cly = u * tri9[:, 1:2] + v * tri9[:, 4:5] + w * tri9[:, 7:8]
    clz = u * tri9[:, 2:3] + v * tri9[:, 5:6] + w * tri9[:, 8:9]
    res_ref[0] = jnp.concatenate([clx - px, cly - py, clz - pz], axis=1)
    ct_ref[0] = tri9


def _run(triangles, points, interpret=False):
    B, F = triangles.shape[0], triangles.shape[1]
    Q = points.shape[1]
    QT = 256
    FT = 512
    tri2 = triangles.reshape(B, F, 9)
    tri = tri2.transpose(0, 2, 1)
    res, ct = pl.pallas_call(
        functools.partial(_sweep_kernel, F=F, FT=FT, QT=QT),
        grid=(B, Q // QT),
        in_specs=[
            pl.BlockSpec((1, 9, F), lambda b, q: (b, 0, 0)),
            pl.BlockSpec((1, F, 9), lambda b, q: (b, 0, 0)),
            pl.BlockSpec((1, QT, 3), lambda b, q: (b, q, 0)),
        ],
        out_specs=[
            pl.BlockSpec((1, QT, 3), lambda b, q: (b, q, 0)),
            pl.BlockSpec((1, QT, 9), lambda b, q: (b, q, 0)),
        ],
        out_shape=[
            jax.ShapeDtypeStruct((B, Q, 3), jnp.float32),
            jax.ShapeDtypeStruct((B, Q, 9), jnp.float32),
        ],
        interpret=interpret,
        compiler_params=None if interpret else pltpu.CompilerParams(dimension_semantics=("arbitrary", "parallel")),
    )(tri, tri2, points)
    return res, ct.reshape(B, Q, 3, 3)


_shard_map = getattr(jax, "shard_map", None)
if _shard_map is None:
    from jax.experimental.shard_map import shard_map as _shard_map


@jax.jit
def _run_sharded(triangles, points):
    B = triangles.shape[0]
    devs = jax.devices()
    n = max(d for d in range(1, min(len(devs), B) + 1) if B % d == 0)
    if n == 1:
        return _run(triangles, points)
    mesh = Mesh(np.array(devs[:n]), ("d",))
    return _shard_map(
        _run,
        mesh=mesh,
        in_specs=(P("d"), P("d")),
        out_specs=(P("d"), P("d")),
        check_vma=False,
    )(triangles, points)


def kernel(triangles, points):
    return _run_sharded(triangles, points)
